# baseline (speedup 1.0000x reference)
import functools

import jax
import jax.numpy as jnp
from jax import lax
from jax.experimental import pallas as pl
from jax.experimental.pallas import tpu as pltpu
from jax.experimental.pallas import tpu_sc as plsc

H = 1080
W = 1920
HW = H * W
N = 2_000_000

NW = 32
PER_TILE = 65536
N_PAD = NW * PER_TILE
NCHUNKS = 4
CHUNK = PER_TILE // NCHUNKS

ROWS = N // 128
ROWS_PAD = N_PAD // 128
BLK = 512
GRID = -(-ROWS // BLK)


def _sc_gather_body(p0, p1, p2, c_hbm, m_hbm, pix_hbm, g0, g1, g2,
                    cv, mv, pv, gv, b0, b1, b2, s0, s1, s2):
    wid = lax.axis_index("s") * 2 + lax.axis_index("c")
    base = wid * PER_TILE

    for j in range(NCHUNKS):
        off = base + j * CHUNK
        pltpu.sync_copy(pix_hbm.at[pl.ds(off, CHUNK)], pv)
        pltpu.sync_copy(c_hbm.at[pl.ds(off, CHUNK)], cv)
        pltpu.sync_copy(m_hbm.at[pl.ds(off, CHUNK)], mv)

        @plsc.parallel_loop(0, CHUNK, step=128, unroll=2)
        def _(i):
            for k in range(8):
                sl = pl.ds(i + k * 16, 16)
                win = cv[sl] > mv[sl]
                gv[sl] = jnp.where(win, pv[sl], 0)

        cp0 = pltpu.async_copy(p0.at[gv], b0, s0)
        cp1 = pltpu.async_copy(p1.at[gv], b1, s1)
        cp2 = pltpu.async_copy(p2.at[gv], b2, s2)
        cp0.wait()
        cp1.wait()
        cp2.wait()
        pltpu.sync_copy(b0, g0.at[pl.ds(off, CHUNK)])
        pltpu.sync_copy(b1, g1.at[pl.ds(off, CHUNK)])
        pltpu.sync_copy(b2, g2.at[pl.ds(off, CHUNK)])


_sc_gather = functools.partial(
    pl.kernel,
    mesh=plsc.VectorSubcoreMesh(core_axis_name="c", subcore_axis_name="s"),
    compiler_params=pltpu.CompilerParams(needs_layout_passes=False),
    out_type=[jax.ShapeDtypeStruct((N_PAD,), jnp.float32)] * 3,
    scratch_types=[
        pltpu.VMEM((CHUNK,), jnp.float32),
        pltpu.VMEM((CHUNK,), jnp.float32),
        pltpu.VMEM((CHUNK,), jnp.int32),
        pltpu.VMEM((CHUNK,), jnp.int32),
        pltpu.VMEM((CHUNK,), jnp.float32),
        pltpu.VMEM((CHUNK,), jnp.float32),
        pltpu.VMEM((CHUNK,), jnp.float32),
        pltpu.SemaphoreType.DMA,
        pltpu.SemaphoreType.DMA,
        pltpu.SemaphoreType.DMA,
    ],
)(_sc_gather_body)


def _ew_body(c_ref, s_ref, m_ref, t_ref, dmin_ref, g0_ref, g1_ref, g2_ref,
             oldt_ref, nmax_ref, ntot_ref, nmin_ref, ncolt_ref):
    c = c_ref[...]
    m = m_ref[...]
    mask = c > m
    nmax_ref[...] = jnp.where(mask, c, m)
    ntot_ref[...] = t_ref[...] + c
    s = s_ref[...]
    d = dmin_ref[...]
    nmin_ref[...] = jnp.where(s < d, s, d)
    ncolt_ref[0] = jnp.where(mask, g0_ref[...], oldt_ref[0])
    ncolt_ref[1] = jnp.where(mask, g1_ref[...], oldt_ref[1])
    ncolt_ref[2] = jnp.where(mask, g2_ref[...], oldt_ref[2])


def _ew_call(c, s, m, t, dmin, g0, g1, g2, oldt):
    flat_spec = pl.BlockSpec((BLK, 128), lambda i: (i, 0))
    col_spec = pl.BlockSpec((3, BLK, 128), lambda i: (0, i, 0))
    return pl.pallas_call(
        _ew_body,
        grid=(GRID,),
        in_specs=[flat_spec] * 8 + [col_spec],
        out_specs=[flat_spec] * 3 + [col_spec],
        out_shape=[
            jax.ShapeDtypeStruct((ROWS, 128), jnp.float32),
            jax.ShapeDtypeStruct((ROWS, 128), jnp.float32),
            jax.ShapeDtypeStruct((ROWS, 128), jnp.float32),
            jax.ShapeDtypeStruct((3, ROWS, 128), jnp.float32),
        ],
    )(c, s, m, t, dmin, g0, g1, g2, oldt)


def kernel(colour, current_gauss_contributions, current_gauss_surface_distances,
           gaussian_max_contribution, gaussian_colours, gaussian_total_contribution,
           gaussian_min_surface_distance, current_gauss_pixels):
    planes = colour.reshape(3, HW)
    padf = jnp.zeros((N_PAD - N,), dtype=jnp.float32)
    padi = jnp.zeros((N_PAD - N,), dtype=jnp.int32)
    g0, g1, g2 = _sc_gather(
        planes[0], planes[1], planes[2],
        jnp.concatenate([current_gauss_contributions, padf]),
        jnp.concatenate([gaussian_max_contribution, padf]),
        jnp.concatenate([current_gauss_pixels, padi]))

    r = lambda x: x.reshape(ROWS, 128)
    rp = lambda x: x.reshape(ROWS_PAD, 128)
    oldt = gaussian_colours.T.reshape(3, ROWS, 128)
    nmax, ntot, nmin, ncolt = _ew_call(
        r(current_gauss_contributions),
        r(current_gauss_surface_distances),
        r(gaussian_max_contribution),
        r(gaussian_total_contribution),
        r(gaussian_min_surface_distance),
        rp(g0), rp(g1), rp(g2), oldt)

    new_colours = ncolt.reshape(3, N).T
    return (nmax.reshape(N), new_colours, ntot.reshape(N), nmin.reshape(N))


# speedup vs baseline: 10.6909x; 10.6909x over previous
import functools

import jax
import jax.numpy as jnp
from jax import lax
from jax.experimental import pallas as pl
from jax.experimental.pallas import tpu as pltpu
from jax.experimental.pallas import tpu_sc as plsc

H = 1080
W = 1920
HW = H * W
N = 2_000_000

NW = 32
PER_TILE = 62528
N_PAD = NW * PER_TILE
CHUNK = 7816
NCHUNKS = PER_TILE // CHUNK

ROWS = N // 128
BLK = 512
GRID = -(-ROWS // BLK)


def _sc_gather_body(p0, p1, p2, idx_hbm, g0, g1, g2,
                    ia, a0, a1, a2, ib, b0, b1, b2,
                    sa0, sa1, sa2, sb0, sb1, sb2):
    wid = lax.axis_index("s") * 2 + lax.axis_index("c")
    base = wid * PER_TILE
    bufs = [(ia, a0, a1, a2, sa0, sa1, sa2), (ib, b0, b1, b2, sb0, sb1, sb2)]

    iv, v0, v1, v2, t0, t1, t2 = bufs[0]
    pltpu.sync_copy(idx_hbm.at[pl.ds(base, CHUNK)], iv)
    inflight = (pltpu.async_copy(p0.at[iv], v0, t0),
                pltpu.async_copy(p1.at[iv], v1, t1),
                pltpu.async_copy(p2.at[iv], v2, t2))
    for j in range(1, NCHUNKS + 1):
        if j < NCHUNKS:
            off = base + j * CHUNK
            iv, v0, v1, v2, t0, t1, t2 = bufs[j % 2]
            pltpu.sync_copy(idx_hbm.at[pl.ds(off, CHUNK)], iv)
            nxt = (pltpu.async_copy(p0.at[iv], v0, t0),
                   pltpu.async_copy(p1.at[iv], v1, t1),
                   pltpu.async_copy(p2.at[iv], v2, t2))
        poff = base + (j - 1) * CHUNK
        _, w0, w1, w2, _, _, _ = bufs[(j - 1) % 2]
        for cp in inflight:
            cp.wait()
        pltpu.sync_copy(w0, g0.at[pl.ds(poff, CHUNK)])
        pltpu.sync_copy(w1, g1.at[pl.ds(poff, CHUNK)])
        pltpu.sync_copy(w2, g2.at[pl.ds(poff, CHUNK)])
        if j < NCHUNKS:
            inflight = nxt


_sc_gather = functools.partial(
    pl.kernel,
    mesh=plsc.VectorSubcoreMesh(core_axis_name="c", subcore_axis_name="s"),
    out_type=[jax.ShapeDtypeStruct((N_PAD,), jnp.float32)] * 3,
    scratch_types=[
        pltpu.VMEM((CHUNK,), jnp.int32),
        pltpu.VMEM((CHUNK,), jnp.float32),
        pltpu.VMEM((CHUNK,), jnp.float32),
        pltpu.VMEM((CHUNK,), jnp.float32),
        pltpu.VMEM((CHUNK,), jnp.int32),
        pltpu.VMEM((CHUNK,), jnp.float32),
        pltpu.VMEM((CHUNK,), jnp.float32),
        pltpu.VMEM((CHUNK,), jnp.float32),
        pltpu.SemaphoreType.DMA,
        pltpu.SemaphoreType.DMA,
        pltpu.SemaphoreType.DMA,
        pltpu.SemaphoreType.DMA,
        pltpu.SemaphoreType.DMA,
        pltpu.SemaphoreType.DMA,
    ],
)(_sc_gather_body)


def _ew_body(c_ref, s_ref, m_ref, t_ref, dmin_ref, g0_ref, g1_ref, g2_ref,
             oldt_ref, nmax_ref, ntot_ref, nmin_ref, ncolt_ref):
    c = c_ref[...]
    m = m_ref[...]
    mask = c > m
    nmax_ref[...] = jnp.where(mask, c, m)
    ntot_ref[...] = t_ref[...] + c
    s = s_ref[...]
    d = dmin_ref[...]
    nmin_ref[...] = jnp.where(s < d, s, d)
    ncolt_ref[0] = jnp.where(mask, g0_ref[...], oldt_ref[0])
    ncolt_ref[1] = jnp.where(mask, g1_ref[...], oldt_ref[1])
    ncolt_ref[2] = jnp.where(mask, g2_ref[...], oldt_ref[2])


def _ew_call(c, s, m, t, dmin, g0, g1, g2, oldt):
    flat_spec = pl.BlockSpec((BLK, 128), lambda i: (i, 0))
    col_spec = pl.BlockSpec((3, BLK, 128), lambda i: (0, i, 0))
    return pl.pallas_call(
        _ew_body,
        grid=(GRID,),
        in_specs=[flat_spec] * 8 + [col_spec],
        out_specs=[flat_spec] * 3 + [col_spec],
        out_shape=[
            jax.ShapeDtypeStruct((ROWS, 128), jnp.float32),
            jax.ShapeDtypeStruct((ROWS, 128), jnp.float32),
            jax.ShapeDtypeStruct((ROWS, 128), jnp.float32),
            jax.ShapeDtypeStruct((3, ROWS, 128), jnp.float32),
        ],
    )(c, s, m, t, dmin, g0, g1, g2, oldt)


def kernel(colour, current_gauss_contributions, current_gauss_surface_distances,
           gaussian_max_contribution, gaussian_colours, gaussian_total_contribution,
           gaussian_min_surface_distance, current_gauss_pixels):
    planes = colour.reshape(3, HW)
    idx_pad = jnp.concatenate(
        [current_gauss_pixels,
         jnp.zeros((N_PAD - N,), dtype=jnp.int32)])
    g0, g1, g2 = _sc_gather(planes[0], planes[1], planes[2], idx_pad)

    r = lambda x: x.reshape(ROWS, 128)
    rp = lambda x: x.reshape(N_PAD // 128, 128)
    oldt = gaussian_colours.T.reshape(3, ROWS, 128)
    nmax, ntot, nmin, ncolt = _ew_call(
        r(current_gauss_contributions),
        r(current_gauss_surface_distances),
        r(gaussian_max_contribution),
        r(gaussian_total_contribution),
        r(gaussian_min_surface_distance),
        rp(g0), rp(g1), rp(g2), oldt)

    new_colours = ncolt.reshape(3, N).T
    return (nmax.reshape(N), new_colours, ntot.reshape(N), nmin.reshape(N))
